# baseline (speedup 1.0000x reference)
import functools

import jax
import jax.numpy as jnp
from jax import lax
from jax.experimental import pallas as pl
from jax.experimental.pallas import tpu as pltpu
from jax.experimental.pallas import tpu_sc as plsc

N = 10000
D = 128
E = 320000

NC = 2
NS = 16
NW = NC * NS
NPAD = 10240
CHUNK = 128
CPT = 79
EPT = CPT * CHUNK
E_PAD = NW * EPT
ROWS_PER_TILE = NPAD // NS

_mesh = plsc.VectorSubcoreMesh(core_axis_name="c", subcore_axis_name="s")


@functools.partial(
    pl.kernel,
    mesh=_mesh,
    out_type=jax.ShapeDtypeStruct((NC, NPAD), jnp.float32),
    scratch_types=[
        pltpu.VMEM((CHUNK,), jnp.int32),
        pltpu.VMEM((CHUNK,), jnp.float32),
        pltpu.VMEM((ROWS_PER_TILE,), jnp.float32),
        pltpu.VMEM_SHARED((NPAD,), jnp.float32),
    ],
)
def _deg_kernel(dst_hbm, degp_hbm, idx_v, ones_v, zbuf_v, deg_sh):
    cid = lax.axis_index("c")
    sid = lax.axis_index("s")
    wid = cid * NS + sid

    for k in range(CHUNK // 16):
        ones_v[pl.ds(k * 16, 16)] = jnp.ones((16,), jnp.float32)
    for k in range(ROWS_PER_TILE // 16):
        zbuf_v[pl.ds(k * 16, 16)] = jnp.zeros((16,), jnp.float32)
    pltpu.sync_copy(zbuf_v, deg_sh.at[pl.ds(sid * ROWS_PER_TILE, ROWS_PER_TILE)])
    plsc.subcore_barrier()

    def body(i, carry):
        base = wid * EPT + i * CHUNK
        pltpu.sync_copy(dst_hbm.at[pl.ds(base, CHUNK)], idx_v)
        pltpu.sync_copy(ones_v, deg_sh.at[idx_v], add=True)
        return carry

    lax.fori_loop(0, CPT, body, 0)
    plsc.subcore_barrier()
    pltpu.sync_copy(
        deg_sh.at[pl.ds(sid * ROWS_PER_TILE, ROWS_PER_TILE)],
        degp_hbm.at[cid, pl.ds(sid * ROWS_PER_TILE, ROWS_PER_TILE)],
    )


@functools.partial(
    pl.kernel,
    mesh=_mesh,
    out_type=jax.ShapeDtypeStruct((NC, NPAD, D), jnp.float32),
    scratch_types=[
        pltpu.VMEM((CHUNK,), jnp.int32),
        pltpu.VMEM((CHUNK,), jnp.int32),
        pltpu.VMEM((CHUNK,), jnp.int32),
        pltpu.VMEM((CHUNK,), jnp.int32),
        pltpu.VMEM((CHUNK, D), jnp.float32),
        pltpu.VMEM((CHUNK, D), jnp.float32),
        pltpu.VMEM_SHARED((NPAD, D), jnp.float32),
        pltpu.SemaphoreType.DMA,
        pltpu.SemaphoreType.DMA,
        pltpu.SemaphoreType.DMA,
    ],
)
def _agg_kernel(src_hbm, dst_hbm, xp_hbm, accp_hbm, idx_s0, idx_d0, idx_s1,
                idx_d1, rows0, rows1, acc_sh, gsem, ssem0, ssem1):
    cid = lax.axis_index("c")
    sid = lax.axis_index("s")
    wid = cid * NS + sid

    def zbody(i, carry):
        for k in range(D // 16):
            rows0[i, pl.ds(k * 16, 16)] = jnp.zeros((16,), jnp.float32)
        return carry

    lax.fori_loop(0, CHUNK, zbody, 0)
    for r in range(ROWS_PER_TILE // CHUNK):
        pltpu.sync_copy(
            rows0, acc_sh.at[pl.ds(sid * ROWS_PER_TILE + r * CHUNK, CHUNK)]
        )
    for k in range(CHUNK // 16):
        idx_d0[pl.ds(k * 16, 16)] = jnp.full((16,), N, jnp.int32)
        idx_d1[pl.ds(k * 16, 16)] = jnp.full((16,), N, jnp.int32)
    plsc.subcore_barrier()

    pltpu.async_copy(rows0, acc_sh.at[idx_d0], ssem0, add=True)
    pltpu.async_copy(rows1, acc_sh.at[idx_d1], ssem1, add=True)

    def step(c, idx_s, idx_d, rows, ssem):
        base = wid * EPT + c * CHUNK
        pltpu.sync_copy(src_hbm.at[pl.ds(base, CHUNK)], idx_s)
        pltpu.make_async_copy(rows, acc_sh.at[idx_d], ssem).wait()
        cp = pltpu.async_copy(xp_hbm.at[idx_s], rows, gsem)
        pltpu.sync_copy(dst_hbm.at[pl.ds(base, CHUNK)], idx_d)
        cp.wait()
        pltpu.async_copy(rows, acc_sh.at[idx_d], ssem, add=True)

    def ebody(j, carry):
        step(2 * j, idx_s0, idx_d0, rows0, ssem0)
        step(2 * j + 1, idx_s1, idx_d1, rows1, ssem1)
        return carry

    lax.fori_loop(0, CPT // 2, ebody, 0)
    step(CPT - 1, idx_s0, idx_d0, rows0, ssem0)
    pltpu.make_async_copy(rows0, acc_sh.at[idx_d0], ssem0).wait()
    pltpu.make_async_copy(rows1, acc_sh.at[idx_d1], ssem1).wait()
    plsc.subcore_barrier()
    pltpu.sync_copy(
        acc_sh.at[pl.ds(sid * ROWS_PER_TILE, ROWS_PER_TILE)],
        accp_hbm.at[cid, pl.ds(sid * ROWS_PER_TILE, ROWS_PER_TILE)],
    )


_BR = 256


def _scale_body(x_r, da_r, db_r, xp_r, x2_r, dis_r):
    deg = da_r[...] + db_r[...] + 1.0
    dis = lax.rsqrt(deg)
    xv = x_r[...]
    xp = xv * dis
    xp_r[...] = xp
    x2_r[...] = xp * dis
    dis_r[...] = dis


def _out_body(a0_r, a1_r, x2_r, dis_r, w_r, b_r, o_r):
    m = (a0_r[...] + a1_r[...]) * dis_r[...] + x2_r[...]
    o_r[...] = (
        jnp.dot(m, w_r[...], preferred_element_type=jnp.float32) + b_r[...]
    )


def kernel(x, edge_index, W, b):
    src = edge_index[0]
    dst = edge_index[1]
    pad_e = E_PAD - E
    pad_idx = jnp.full((pad_e,), N, jnp.int32)
    src_p = jnp.concatenate([src, pad_idx])
    dst_p = jnp.concatenate([dst, pad_idx])
    x_p = jnp.pad(x, ((0, NPAD - N), (0, 0)))

    degp = _deg_kernel(dst_p)
    dega = degp[0].reshape(NPAD, 1)
    degb = degp[1].reshape(NPAD, 1)

    grid = NPAD // _BR
    row_spec = pl.BlockSpec((_BR, D), lambda i: (i, 0))
    col_spec = pl.BlockSpec((_BR, 1), lambda i: (i, 0))
    xp, x2, dis = pl.pallas_call(
        _scale_body,
        grid=(grid,),
        in_specs=[row_spec, col_spec, col_spec],
        out_specs=[row_spec, row_spec, col_spec],
        out_shape=[
            jax.ShapeDtypeStruct((NPAD, D), jnp.float32),
            jax.ShapeDtypeStruct((NPAD, D), jnp.float32),
            jax.ShapeDtypeStruct((NPAD, 1), jnp.float32),
        ],
    )(x_p, dega, degb)

    accp = _agg_kernel(src_p, dst_p, xp)

    o = pl.pallas_call(
        _out_body,
        grid=(grid,),
        in_specs=[
            row_spec,
            row_spec,
            row_spec,
            col_spec,
            pl.BlockSpec((D, D), lambda i: (0, 0)),
            pl.BlockSpec((1, D), lambda i: (0, 0)),
        ],
        out_specs=row_spec,
        out_shape=jax.ShapeDtypeStruct((NPAD, D), jnp.float32),
    )(accp[0], accp[1], x2, dis, W, b.reshape(1, D))

    return o[:N]


# speedup vs baseline: 1.0139x; 1.0139x over previous
import functools

import jax
import jax.numpy as jnp
from jax import lax
from jax.experimental import pallas as pl
from jax.experimental.pallas import tpu as pltpu
from jax.experimental.pallas import tpu_sc as plsc

N = 10000
D = 128
E = 320000

NC = 2
NS = 16
NW = NC * NS
NPAD = 10240
CHUNK = 128
CPT = 79
EPT = CPT * CHUNK
E_PAD = NW * EPT
ROWS_PER_TILE = NPAD // NS

_mesh = plsc.VectorSubcoreMesh(core_axis_name="c", subcore_axis_name="s")


@functools.partial(
    pl.kernel,
    mesh=_mesh,
    out_type=jax.ShapeDtypeStruct((NC, NPAD), jnp.float32),
    scratch_types=[
        pltpu.VMEM((CHUNK,), jnp.int32),
        pltpu.VMEM((CHUNK,), jnp.int32),
        pltpu.VMEM((CHUNK,), jnp.float32),
        pltpu.VMEM((ROWS_PER_TILE,), jnp.float32),
        pltpu.VMEM_SHARED((NPAD,), jnp.float32),
        pltpu.SemaphoreType.DMA,
        pltpu.SemaphoreType.DMA,
    ],
)
def _deg_kernel(dst_hbm, degp_hbm, idx0, idx1, ones_v, zbuf_v, deg_sh,
                dsem0, dsem1):
    cid = lax.axis_index("c")
    sid = lax.axis_index("s")
    wid = cid * NS + sid

    for k in range(CHUNK // 16):
        ones_v[pl.ds(k * 16, 16)] = jnp.ones((16,), jnp.float32)
        idx0[pl.ds(k * 16, 16)] = jnp.full((16,), N, jnp.int32)
        idx1[pl.ds(k * 16, 16)] = jnp.full((16,), N, jnp.int32)
    for k in range(ROWS_PER_TILE // 16):
        zbuf_v[pl.ds(k * 16, 16)] = jnp.zeros((16,), jnp.float32)
    pltpu.sync_copy(zbuf_v, deg_sh.at[pl.ds(sid * ROWS_PER_TILE, ROWS_PER_TILE)])
    plsc.subcore_barrier()

    pltpu.async_copy(ones_v, deg_sh.at[idx0], dsem0, add=True)
    pltpu.async_copy(ones_v, deg_sh.at[idx1], dsem1, add=True)

    def step(c, idx, dsem):
        base = wid * EPT + c * CHUNK
        pltpu.make_async_copy(ones_v, deg_sh.at[idx], dsem).wait()
        pltpu.sync_copy(dst_hbm.at[pl.ds(base, CHUNK)], idx)
        pltpu.async_copy(ones_v, deg_sh.at[idx], dsem, add=True)

    def body(j, carry):
        step(2 * j, idx0, dsem0)
        step(2 * j + 1, idx1, dsem1)
        return carry

    lax.fori_loop(0, CPT // 2, body, 0)
    step(CPT - 1, idx0, dsem0)
    pltpu.make_async_copy(ones_v, deg_sh.at[idx0], dsem0).wait()
    pltpu.make_async_copy(ones_v, deg_sh.at[idx1], dsem1).wait()
    plsc.subcore_barrier()
    pltpu.sync_copy(
        deg_sh.at[pl.ds(sid * ROWS_PER_TILE, ROWS_PER_TILE)],
        degp_hbm.at[cid, pl.ds(sid * ROWS_PER_TILE, ROWS_PER_TILE)],
    )


@functools.partial(
    pl.kernel,
    mesh=_mesh,
    out_type=jax.ShapeDtypeStruct((NC, NPAD, D), jnp.float32),
    scratch_types=[
        pltpu.VMEM((CHUNK,), jnp.int32),
        pltpu.VMEM((CHUNK,), jnp.int32),
        pltpu.VMEM((CHUNK,), jnp.int32),
        pltpu.VMEM((CHUNK,), jnp.int32),
        pltpu.VMEM((CHUNK, D), jnp.float32),
        pltpu.VMEM((CHUNK, D), jnp.float32),
        pltpu.VMEM_SHARED((NPAD, D), jnp.float32),
        pltpu.SemaphoreType.DMA,
        pltpu.SemaphoreType.DMA,
        pltpu.SemaphoreType.DMA,
    ],
)
def _agg_kernel(src_hbm, dst_hbm, xp_hbm, accp_hbm, idx_s0, idx_d0, idx_s1,
                idx_d1, rows0, rows1, acc_sh, gsem, ssem0, ssem1):
    cid = lax.axis_index("c")
    sid = lax.axis_index("s")
    wid = cid * NS + sid

    def zbody(i, carry):
        for k in range(D // 16):
            rows0[i, pl.ds(k * 16, 16)] = jnp.zeros((16,), jnp.float32)
        return carry

    lax.fori_loop(0, CHUNK, zbody, 0)
    for r in range(ROWS_PER_TILE // CHUNK):
        pltpu.sync_copy(
            rows0, acc_sh.at[pl.ds(sid * ROWS_PER_TILE + r * CHUNK, CHUNK)]
        )
    for k in range(CHUNK // 16):
        idx_d0[pl.ds(k * 16, 16)] = jnp.full((16,), N, jnp.int32)
        idx_d1[pl.ds(k * 16, 16)] = jnp.full((16,), N, jnp.int32)
    plsc.subcore_barrier()

    pltpu.async_copy(rows0, acc_sh.at[idx_d0], ssem0, add=True)
    pltpu.async_copy(rows1, acc_sh.at[idx_d1], ssem1, add=True)

    def step(c, idx_s, idx_d, rows, ssem):
        base = wid * EPT + c * CHUNK
        pltpu.sync_copy(src_hbm.at[pl.ds(base, CHUNK)], idx_s)
        pltpu.make_async_copy(rows, acc_sh.at[idx_d], ssem).wait()
        cp = pltpu.async_copy(xp_hbm.at[idx_s], rows, gsem)
        pltpu.sync_copy(dst_hbm.at[pl.ds(base, CHUNK)], idx_d)
        cp.wait()
        pltpu.async_copy(rows, acc_sh.at[idx_d], ssem, add=True)

    def ebody(j, carry):
        step(2 * j, idx_s0, idx_d0, rows0, ssem0)
        step(2 * j + 1, idx_s1, idx_d1, rows1, ssem1)
        return carry

    lax.fori_loop(0, CPT // 2, ebody, 0)
    step(CPT - 1, idx_s0, idx_d0, rows0, ssem0)
    pltpu.make_async_copy(rows0, acc_sh.at[idx_d0], ssem0).wait()
    pltpu.make_async_copy(rows1, acc_sh.at[idx_d1], ssem1).wait()
    plsc.subcore_barrier()
    pltpu.sync_copy(
        acc_sh.at[pl.ds(sid * ROWS_PER_TILE, ROWS_PER_TILE)],
        accp_hbm.at[cid, pl.ds(sid * ROWS_PER_TILE, ROWS_PER_TILE)],
    )


_BR = 256


def _scale_body(x_r, da_r, db_r, xp_r, x2_r, dis_r):
    deg = da_r[...] + db_r[...] + 1.0
    dis = lax.rsqrt(deg)
    xv = x_r[...]
    xp = xv * dis
    xp_r[...] = xp
    x2_r[...] = xp * dis
    dis_r[...] = dis


def _out_body(a0_r, a1_r, x2_r, dis_r, w_r, b_r, o_r):
    m = (a0_r[...] + a1_r[...]) * dis_r[...] + x2_r[...]
    o_r[...] = (
        jnp.dot(m, w_r[...], preferred_element_type=jnp.float32) + b_r[...]
    )


def kernel(x, edge_index, W, b):
    src = edge_index[0]
    dst = edge_index[1]
    pad_e = E_PAD - E
    pad_idx = jnp.full((pad_e,), N, jnp.int32)
    src_p = jnp.concatenate([src, pad_idx])
    dst_p = jnp.concatenate([dst, pad_idx])
    x_p = jnp.pad(x, ((0, NPAD - N), (0, 0)))

    degp = _deg_kernel(dst_p)
    dega = degp[0].reshape(NPAD, 1)
    degb = degp[1].reshape(NPAD, 1)

    grid = NPAD // _BR
    row_spec = pl.BlockSpec((_BR, D), lambda i: (i, 0))
    col_spec = pl.BlockSpec((_BR, 1), lambda i: (i, 0))
    xp, x2, dis = pl.pallas_call(
        _scale_body,
        grid=(grid,),
        in_specs=[row_spec, col_spec, col_spec],
        out_specs=[row_spec, row_spec, col_spec],
        out_shape=[
            jax.ShapeDtypeStruct((NPAD, D), jnp.float32),
            jax.ShapeDtypeStruct((NPAD, D), jnp.float32),
            jax.ShapeDtypeStruct((NPAD, 1), jnp.float32),
        ],
    )(x_p, dega, degb)

    accp = _agg_kernel(src_p, dst_p, xp)

    o = pl.pallas_call(
        _out_body,
        grid=(grid,),
        in_specs=[
            row_spec,
            row_spec,
            row_spec,
            col_spec,
            pl.BlockSpec((D, D), lambda i: (0, 0)),
            pl.BlockSpec((1, D), lambda i: (0, 0)),
        ],
        out_specs=row_spec,
        out_shape=jax.ShapeDtypeStruct((NPAD, D), jnp.float32),
    )(accp[0], accp[1], x2, dis, W, b.reshape(1, D))

    return o[:N]
